# baseline (speedup 1.0000x reference)
import functools

import jax
import jax.numpy as jnp
from jax.experimental import pallas as pl
from jax.experimental.pallas import tpu as pltpu

_GS = 4
_G = _GS * _GS * _GS
_TK = 8
_TM = 2048
_TB = 64
_NCLS = 18


def _knn_conv1_body(nseed, tq, kt_steps, wg_ref, rel_ref, sxyz_ref, feats_ref,
                    w1t_ref, b1_ref, h1_ref, stats_ref, acc_scr, rw_scr):
    b = pl.program_id(0)
    kt = pl.program_id(1)

    @pl.when((b == 0) & (kt == 0))
    def _():
        stats_ref[:, :] = jnp.zeros_like(stats_ref)

    p = jnp.mod(kt - 1, 2)
    interp = jnp.dot(acc_scr[p], feats_ref[0],
                     preferred_element_type=jnp.float32,
                     precision=jax.lax.Precision.HIGHEST)
    interp = interp * rw_scr[p]
    xcat = jnp.concatenate([rel_ref[0], interp], axis=1)
    y = jnp.dot(xcat, w1t_ref[...], preferred_element_type=jnp.float32)
    y = y + b1_ref[:, :]
    h1_ref[0] = y

    @pl.when(kt > 0)
    def _():
        stats_ref[0:1, :] += jnp.sum(y, axis=0, keepdims=True)
        stats_ref[1:2, :] += jnp.sum(y * y, axis=0, keepdims=True)

    wg = wg_ref[0]
    s = sxyz_ref[0]

    dx = wg[:, 0:1] - s[0:1, :]
    d2 = dx * dx
    dy = wg[:, 1:2] - s[1:2, :]
    d2 = d2 + dy * dy
    dz = wg[:, 2:3] - s[2:3, :]
    d2 = d2 + dz * dz

    acc = None
    wsum = None
    d2w = d2
    for _ in range(3):
        m = jnp.min(d2w, axis=1, keepdims=True)
        sel = d2w == m
        w = 1.0 / (jnp.sqrt(m) + 1e-8)
        acc = jnp.where(sel, w, 0.0 if acc is None else acc)
        wsum = w if wsum is None else wsum + w
        d2w = jnp.where(sel, jnp.float32(jnp.inf), d2w)

    acc_scr[jnp.mod(kt, 2)] = acc
    rw_scr[jnp.mod(kt, 2)] = 1.0 / wsum


def _bn_relu_mm_body(x_ref, sc_ref, sh_ref, wt_ref, b_ref, y_ref, stats_ref):
    i = pl.program_id(0)
    h = jnp.maximum(x_ref[...] * sc_ref[:, :] + sh_ref[:, :], 0.0)
    y = jnp.dot(h, wt_ref[...], preferred_element_type=jnp.float32)
    y = y + b_ref[:, :]
    y_ref[...] = y

    @pl.when(i == 0)
    def _():
        stats_ref[:, :] = jnp.zeros_like(stats_ref)

    stats_ref[0:1, :] += jnp.sum(y, axis=0, keepdims=True)
    stats_ref[1:2, :] += jnp.sum(y * y, axis=0, keepdims=True)


def _pool_conv_body(x_ref, sc_ref, sh_ref, wt_ref, b_ref, y_ref, stats_ref):
    i = pl.program_id(0)
    h = jnp.maximum(x_ref[...] * sc_ref[:, :] + sh_ref[:, :], 0.0)
    pooled = jnp.max(h, axis=1)
    y = jnp.dot(pooled, wt_ref[...], preferred_element_type=jnp.float32)
    y = y + b_ref[:, :]
    y_ref[...] = y

    @pl.when(i == 0)
    def _():
        stats_ref[:, :] = jnp.zeros_like(stats_ref)

    stats_ref[0:1, :] += jnp.sum(y, axis=0, keepdims=True)
    stats_ref[1:2, :] += jnp.sum(y * y, axis=0, keepdims=True)


def _head_body(x_ref, sc1_ref, sh1_ref, w2t_ref, b2_ref, g2_ref, be2_ref,
               w3t_ref, b3_ref, y_ref):
    h = jnp.maximum(x_ref[...] * sc1_ref[:, :] + sh1_ref[:, :], 0.0)
    c2 = jnp.dot(h, w2t_ref[...], preferred_element_type=jnp.float32)
    c2 = c2 + b2_ref[:, :]
    m = jnp.mean(c2, axis=0, keepdims=True)
    v = jnp.mean(c2 * c2, axis=0, keepdims=True) - m * m
    h2 = (c2 - m) / jnp.sqrt(v + 1e-5) * g2_ref[:, :] + be2_ref[:, :]
    h2 = jnp.maximum(h2, 0.0)
    y = jnp.dot(h2, w3t_ref[...], preferred_element_type=jnp.float32)
    y_ref[...] = y + b3_ref[:, :]


def _affine(stats, count, g, be, eps=1e-5):
    mean = stats[0] / count
    var = stats[1] / count - mean * mean
    sc = g / jnp.sqrt(var + eps)
    sh = be - mean * sc
    return sc.reshape(1, -1), sh.reshape(1, -1)


def kernel(center, size, heading, seed_xyz, seed_features, W1, b1, g1, be1,
           W2, b2, g2, be2, W3, b3, g3, be3, Wc1, bc1, gc1, bec1,
           Wc2, bc2, gc2, bec2, Wc3, bc3):
    B, K = size.shape[:2]
    N = seed_xyz.shape[1]
    C = seed_features.shape[1]
    Q = K * _G
    TQ = _TK * _G
    F = W1.shape[0]

    grid_step = jnp.linspace(-1.0, 1.0, _GS)
    gx = jnp.broadcast_to(grid_step.reshape(_GS, 1, 1), (_GS, _GS, _GS)).reshape(1, 1, _G)
    gy = jnp.broadcast_to(grid_step.reshape(1, _GS, 1), (_GS, _GS, _GS)).reshape(1, 1, _G)
    gz = jnp.broadcast_to(grid_step.reshape(1, 1, _GS), (_GS, _GS, _GS)).reshape(1, 1, _G)
    x_grid = gx * size[:, :, 0:1]
    y_grid = gy * size[:, :, 1:2]
    z_grid = gz * size[:, :, 2:3]
    whole_grid = jnp.stack([x_grid, y_grid, z_grid], axis=-1)
    c = jnp.cos(heading)
    s = jnp.sin(heading)
    z0 = jnp.zeros_like(c)
    o1 = jnp.ones_like(c)
    rot = jnp.stack([jnp.stack([c, -s, z0], axis=-1),
                     jnp.stack([s, c, z0], axis=-1),
                     jnp.stack([z0, z0, o1], axis=-1)], axis=-2)
    rel_grid = jnp.einsum('bkgi,bkji->bkgj', whole_grid, rot)
    whole_grid = rel_grid + center[:, :, None, :]
    wg = whole_grid.reshape(B, Q, 3)
    rel = rel_grid.reshape(B, Q, 3)

    sxyz_t = jnp.transpose(seed_xyz, (0, 2, 1))
    feats_t = jnp.transpose(seed_features, (0, 2, 1))
    w1t = jnp.transpose(W1, (1, 0))

    kt_steps = K // _TK
    h1, stats1 = pl.pallas_call(
        functools.partial(_knn_conv1_body, N, TQ, kt_steps),
        grid=(B, kt_steps + 1),
        in_specs=[
            pl.BlockSpec((1, TQ, 3), lambda b, k: (b, jnp.minimum(k, kt_steps - 1), 0)),
            pl.BlockSpec((1, TQ, 3), lambda b, k: (b, jnp.maximum(k - 1, 0), 0)),
            pl.BlockSpec((1, 3, N), lambda b, k: (b, 0, 0)),
            pl.BlockSpec((1, N, C), lambda b, k: (b, 0, 0)),
            pl.BlockSpec((3 + C, F), lambda b, k: (0, 0)),
            pl.BlockSpec((1, F), lambda b, k: (0, 0)),
        ],
        out_specs=[
            pl.BlockSpec((1, TQ, F), lambda b, k: (b, jnp.maximum(k - 1, 0), 0)),
            pl.BlockSpec((2, F), lambda b, k: (0, 0)),
        ],
        out_shape=[
            jax.ShapeDtypeStruct((B, Q, F), jnp.float32),
            jax.ShapeDtypeStruct((2, F), jnp.float32),
        ],
        scratch_shapes=[
            pltpu.VMEM((2, TQ, N), jnp.float32),
            pltpu.VMEM((2, TQ, 1), jnp.float32),
        ],
        compiler_params=pltpu.CompilerParams(
            dimension_semantics=("arbitrary", "arbitrary")),
    )(wg, rel, sxyz_t, feats_t, w1t, b1.reshape(1, F))

    BQ = B * Q
    h1 = h1.reshape(BQ, F)
    sc1, sh1 = _affine(stats1, BQ, g1, be1)

    def bn_relu_mm(x, sc, sh, wt, bias):
        rows = x.shape[0]
        return pl.pallas_call(
            _bn_relu_mm_body,
            grid=(rows // _TM,),
            in_specs=[
                pl.BlockSpec((_TM, F), lambda i: (i, 0)),
                pl.BlockSpec((1, F), lambda i: (0, 0)),
                pl.BlockSpec((1, F), lambda i: (0, 0)),
                pl.BlockSpec((F, F), lambda i: (0, 0)),
                pl.BlockSpec((1, F), lambda i: (0, 0)),
            ],
            out_specs=[
                pl.BlockSpec((_TM, F), lambda i: (i, 0)),
                pl.BlockSpec((2, F), lambda i: (0, 0)),
            ],
            out_shape=[
                jax.ShapeDtypeStruct((rows, F), jnp.float32),
                jax.ShapeDtypeStruct((2, F), jnp.float32),
            ],
            compiler_params=pltpu.CompilerParams(
                dimension_semantics=("arbitrary",)),
        )(x, sc, sh, wt, bias)

    h2, stats2 = bn_relu_mm(h1, sc1, sh1, jnp.transpose(W2, (1, 0)),
                            b2.reshape(1, F))
    sc2, sh2 = _affine(stats2, BQ, g2, be2)
    h3, stats3 = bn_relu_mm(h2, sc2, sh2, jnp.transpose(W3, (1, 0)),
                            b3.reshape(1, F))
    sc3, sh3 = _affine(stats3, BQ, g3, be3)

    BK = B * K
    h3g = h3.reshape(BK, _G, F)
    c1, statsc1 = pl.pallas_call(
        _pool_conv_body,
        grid=(BK // _TB,),
        in_specs=[
            pl.BlockSpec((_TB, _G, F), lambda i: (i, 0, 0)),
            pl.BlockSpec((1, F), lambda i: (0, 0)),
            pl.BlockSpec((1, F), lambda i: (0, 0)),
            pl.BlockSpec((F, F), lambda i: (0, 0)),
            pl.BlockSpec((1, F), lambda i: (0, 0)),
        ],
        out_specs=[
            pl.BlockSpec((_TB, F), lambda i: (i, 0)),
            pl.BlockSpec((2, F), lambda i: (0, 0)),
        ],
        out_shape=[
            jax.ShapeDtypeStruct((BK, F), jnp.float32),
            jax.ShapeDtypeStruct((2, F), jnp.float32),
        ],
        compiler_params=pltpu.CompilerParams(
            dimension_semantics=("arbitrary",)),
    )(h3g, sc3, sh3, jnp.transpose(Wc1, (1, 0)), bc1.reshape(1, F))

    scc1, shc1 = _affine(statsc1, BK, gc1, bec1)

    w3t = jnp.transpose(Wc3[-_NCLS:], (1, 0))
    out = pl.pallas_call(
        _head_body,
        grid=(1,),
        in_specs=[
            pl.BlockSpec((BK, F), lambda i: (0, 0)),
            pl.BlockSpec((1, F), lambda i: (0, 0)),
            pl.BlockSpec((1, F), lambda i: (0, 0)),
            pl.BlockSpec((F, F), lambda i: (0, 0)),
            pl.BlockSpec((1, F), lambda i: (0, 0)),
            pl.BlockSpec((1, F), lambda i: (0, 0)),
            pl.BlockSpec((1, F), lambda i: (0, 0)),
            pl.BlockSpec((F, _NCLS), lambda i: (0, 0)),
            pl.BlockSpec((1, _NCLS), lambda i: (0, 0)),
        ],
        out_specs=pl.BlockSpec((BK, _NCLS), lambda i: (0, 0)),
        out_shape=jax.ShapeDtypeStruct((BK, _NCLS), jnp.float32),
    )(c1, scc1, shc1, jnp.transpose(Wc2, (1, 0)), bc2.reshape(1, F),
      gc2.reshape(1, F), bec2.reshape(1, F), w3t, bc3[-_NCLS:].reshape(1, _NCLS))

    return out.reshape(B, K, _NCLS)


# speedup vs baseline: 1.1001x; 1.1001x over previous
import functools

import jax
import jax.numpy as jnp
from jax.experimental import pallas as pl
from jax.experimental.pallas import tpu as pltpu

_GS = 4
_G = _GS * _GS * _GS
_TK = 16
_TM = 2048
_TB = 64
_NCLS = 18


def _knn_conv1_body(nseed, tq, wg_ref, rel_ref, sxyz_ref, feats_ref,
                    w1t_ref, b1_ref, h1_ref, stats_ref):
    b = pl.program_id(0)
    kt = pl.program_id(1)

    wg = wg_ref[0]
    rel = rel_ref[0]
    s = sxyz_ref[0]

    dx = wg[:, 0:1] - s[0:1, :]
    d2 = dx * dx
    dy = wg[:, 1:2] - s[1:2, :]
    d2 = d2 + dy * dy
    dz = wg[:, 2:3] - s[2:3, :]
    d2 = d2 + dz * dz

    acc = None
    wsum = None
    d2w = d2
    for _ in range(3):
        m = jnp.min(d2w, axis=1, keepdims=True)
        sel = d2w == m
        w = 1.0 / (jnp.sqrt(m) + 1e-8)
        acc = jnp.where(sel, w, 0.0 if acc is None else acc)
        wsum = w if wsum is None else wsum + w
        d2w = jnp.where(sel, jnp.float32(jnp.inf), d2w)

    interp = jnp.dot(acc, feats_ref[0], preferred_element_type=jnp.float32,
                     precision=jax.lax.Precision.HIGHEST)
    interp = interp * (1.0 / wsum)
    xcat = jnp.concatenate([rel, interp], axis=1)
    y = jnp.dot(xcat, w1t_ref[...], preferred_element_type=jnp.float32)
    y = y + b1_ref[:, :]
    h1_ref[0] = y

    @pl.when((b == 0) & (kt == 0))
    def _():
        stats_ref[:, :] = jnp.zeros_like(stats_ref)

    stats_ref[0:1, :] += jnp.sum(y, axis=0, keepdims=True)
    stats_ref[1:2, :] += jnp.sum(y * y, axis=0, keepdims=True)


def _bn_relu_mm_body(x_ref, sc_ref, sh_ref, wt_ref, b_ref, y_ref, stats_ref):
    i = pl.program_id(0)
    h = jnp.maximum(x_ref[...] * sc_ref[:, :] + sh_ref[:, :], 0.0)
    y = jnp.dot(h, wt_ref[...], preferred_element_type=jnp.float32)
    y = y + b_ref[:, :]
    y_ref[...] = y

    @pl.when(i == 0)
    def _():
        stats_ref[:, :] = jnp.zeros_like(stats_ref)

    stats_ref[0:1, :] += jnp.sum(y, axis=0, keepdims=True)
    stats_ref[1:2, :] += jnp.sum(y * y, axis=0, keepdims=True)


def _pool_conv_body(x_ref, sc_ref, sh_ref, wt_ref, b_ref, y_ref, stats_ref):
    i = pl.program_id(0)
    h = jnp.maximum(x_ref[...] * sc_ref[:, :] + sh_ref[:, :], 0.0)
    pooled = jnp.max(h, axis=1)
    y = jnp.dot(pooled, wt_ref[...], preferred_element_type=jnp.float32)
    y = y + b_ref[:, :]
    y_ref[...] = y

    @pl.when(i == 0)
    def _():
        stats_ref[:, :] = jnp.zeros_like(stats_ref)

    stats_ref[0:1, :] += jnp.sum(y, axis=0, keepdims=True)
    stats_ref[1:2, :] += jnp.sum(y * y, axis=0, keepdims=True)


def _head_body(x_ref, sc1_ref, sh1_ref, w2t_ref, b2_ref, g2_ref, be2_ref,
               w3t_ref, b3_ref, y_ref):
    h = jnp.maximum(x_ref[...] * sc1_ref[:, :] + sh1_ref[:, :], 0.0)
    c2 = jnp.dot(h, w2t_ref[...], preferred_element_type=jnp.float32)
    c2 = c2 + b2_ref[:, :]
    m = jnp.mean(c2, axis=0, keepdims=True)
    v = jnp.mean(c2 * c2, axis=0, keepdims=True) - m * m
    h2 = (c2 - m) / jnp.sqrt(v + 1e-5) * g2_ref[:, :] + be2_ref[:, :]
    h2 = jnp.maximum(h2, 0.0)
    y = jnp.dot(h2, w3t_ref[...], preferred_element_type=jnp.float32)
    y_ref[...] = y + b3_ref[:, :]


def _affine(stats, count, g, be, eps=1e-5):
    mean = stats[0] / count
    var = stats[1] / count - mean * mean
    sc = g / jnp.sqrt(var + eps)
    sh = be - mean * sc
    return sc.reshape(1, -1), sh.reshape(1, -1)


def kernel(center, size, heading, seed_xyz, seed_features, W1, b1, g1, be1,
           W2, b2, g2, be2, W3, b3, g3, be3, Wc1, bc1, gc1, bec1,
           Wc2, bc2, gc2, bec2, Wc3, bc3):
    B, K = size.shape[:2]
    N = seed_xyz.shape[1]
    C = seed_features.shape[1]
    Q = K * _G
    TQ = _TK * _G
    F = W1.shape[0]

    grid_step = jnp.linspace(-1.0, 1.0, _GS)
    gx = jnp.broadcast_to(grid_step.reshape(_GS, 1, 1), (_GS, _GS, _GS)).reshape(1, 1, _G)
    gy = jnp.broadcast_to(grid_step.reshape(1, _GS, 1), (_GS, _GS, _GS)).reshape(1, 1, _G)
    gz = jnp.broadcast_to(grid_step.reshape(1, 1, _GS), (_GS, _GS, _GS)).reshape(1, 1, _G)
    x_grid = gx * size[:, :, 0:1]
    y_grid = gy * size[:, :, 1:2]
    z_grid = gz * size[:, :, 2:3]
    whole_grid = jnp.stack([x_grid, y_grid, z_grid], axis=-1)
    c = jnp.cos(heading)
    s = jnp.sin(heading)
    z0 = jnp.zeros_like(c)
    o1 = jnp.ones_like(c)
    rot = jnp.stack([jnp.stack([c, -s, z0], axis=-1),
                     jnp.stack([s, c, z0], axis=-1),
                     jnp.stack([z0, z0, o1], axis=-1)], axis=-2)
    rel_grid = jnp.einsum('bkgi,bkji->bkgj', whole_grid, rot)
    whole_grid = rel_grid + center[:, :, None, :]
    wg = whole_grid.reshape(B, Q, 3)
    rel = rel_grid.reshape(B, Q, 3)

    sxyz_t = jnp.transpose(seed_xyz, (0, 2, 1))
    feats_t = jnp.transpose(seed_features, (0, 2, 1))
    w1t = jnp.transpose(W1, (1, 0))

    kt_steps = K // _TK
    h1, stats1 = pl.pallas_call(
        functools.partial(_knn_conv1_body, N, TQ),
        grid=(B, kt_steps),
        in_specs=[
            pl.BlockSpec((1, TQ, 3), lambda b, k: (b, k, 0)),
            pl.BlockSpec((1, TQ, 3), lambda b, k: (b, k, 0)),
            pl.BlockSpec((1, 3, N), lambda b, k: (b, 0, 0)),
            pl.BlockSpec((1, N, C), lambda b, k: (b, 0, 0)),
            pl.BlockSpec((3 + C, F), lambda b, k: (0, 0)),
            pl.BlockSpec((1, F), lambda b, k: (0, 0)),
        ],
        out_specs=[
            pl.BlockSpec((1, TQ, F), lambda b, k: (b, k, 0)),
            pl.BlockSpec((2, F), lambda b, k: (0, 0)),
        ],
        out_shape=[
            jax.ShapeDtypeStruct((B, Q, F), jnp.float32),
            jax.ShapeDtypeStruct((2, F), jnp.float32),
        ],
        compiler_params=pltpu.CompilerParams(
            dimension_semantics=("arbitrary", "arbitrary")),
    )(wg, rel, sxyz_t, feats_t, w1t, b1.reshape(1, F))

    BQ = B * Q
    h1 = h1.reshape(BQ, F)
    sc1, sh1 = _affine(stats1, BQ, g1, be1)

    def bn_relu_mm(x, sc, sh, wt, bias):
        rows = x.shape[0]
        return pl.pallas_call(
            _bn_relu_mm_body,
            grid=(rows // _TM,),
            in_specs=[
                pl.BlockSpec((_TM, F), lambda i: (i, 0)),
                pl.BlockSpec((1, F), lambda i: (0, 0)),
                pl.BlockSpec((1, F), lambda i: (0, 0)),
                pl.BlockSpec((F, F), lambda i: (0, 0)),
                pl.BlockSpec((1, F), lambda i: (0, 0)),
            ],
            out_specs=[
                pl.BlockSpec((_TM, F), lambda i: (i, 0)),
                pl.BlockSpec((2, F), lambda i: (0, 0)),
            ],
            out_shape=[
                jax.ShapeDtypeStruct((rows, F), jnp.float32),
                jax.ShapeDtypeStruct((2, F), jnp.float32),
            ],
            compiler_params=pltpu.CompilerParams(
                dimension_semantics=("arbitrary",)),
        )(x, sc, sh, wt, bias)

    h2, stats2 = bn_relu_mm(h1, sc1, sh1, jnp.transpose(W2, (1, 0)),
                            b2.reshape(1, F))
    sc2, sh2 = _affine(stats2, BQ, g2, be2)
    h3, stats3 = bn_relu_mm(h2, sc2, sh2, jnp.transpose(W3, (1, 0)),
                            b3.reshape(1, F))
    sc3, sh3 = _affine(stats3, BQ, g3, be3)

    BK = B * K
    h3g = h3.reshape(BK, _G, F)
    c1, statsc1 = pl.pallas_call(
        _pool_conv_body,
        grid=(BK // _TB,),
        in_specs=[
            pl.BlockSpec((_TB, _G, F), lambda i: (i, 0, 0)),
            pl.BlockSpec((1, F), lambda i: (0, 0)),
            pl.BlockSpec((1, F), lambda i: (0, 0)),
            pl.BlockSpec((F, F), lambda i: (0, 0)),
            pl.BlockSpec((1, F), lambda i: (0, 0)),
        ],
        out_specs=[
            pl.BlockSpec((_TB, F), lambda i: (i, 0)),
            pl.BlockSpec((2, F), lambda i: (0, 0)),
        ],
        out_shape=[
            jax.ShapeDtypeStruct((BK, F), jnp.float32),
            jax.ShapeDtypeStruct((2, F), jnp.float32),
        ],
        compiler_params=pltpu.CompilerParams(
            dimension_semantics=("arbitrary",)),
    )(h3g, sc3, sh3, jnp.transpose(Wc1, (1, 0)), bc1.reshape(1, F))

    scc1, shc1 = _affine(statsc1, BK, gc1, bec1)

    w3t = jnp.transpose(Wc3[-_NCLS:], (1, 0))
    out = pl.pallas_call(
        _head_body,
        grid=(1,),
        in_specs=[
            pl.BlockSpec((BK, F), lambda i: (0, 0)),
            pl.BlockSpec((1, F), lambda i: (0, 0)),
            pl.BlockSpec((1, F), lambda i: (0, 0)),
            pl.BlockSpec((F, F), lambda i: (0, 0)),
            pl.BlockSpec((1, F), lambda i: (0, 0)),
            pl.BlockSpec((1, F), lambda i: (0, 0)),
            pl.BlockSpec((1, F), lambda i: (0, 0)),
            pl.BlockSpec((F, _NCLS), lambda i: (0, 0)),
            pl.BlockSpec((1, _NCLS), lambda i: (0, 0)),
        ],
        out_specs=pl.BlockSpec((BK, _NCLS), lambda i: (0, 0)),
        out_shape=jax.ShapeDtypeStruct((BK, _NCLS), jnp.float32),
    )(c1, scc1, shc1, jnp.transpose(Wc2, (1, 0)), bc2.reshape(1, F),
      gc2.reshape(1, F), bec2.reshape(1, F), w3t, bc3[-_NCLS:].reshape(1, _NCLS))

    return out.reshape(B, K, _NCLS)


# speedup vs baseline: 1.1038x; 1.0033x over previous
import functools

import jax
import jax.numpy as jnp
from jax.experimental import pallas as pl
from jax.experimental.pallas import tpu as pltpu

_GS = 4
_G = _GS * _GS * _GS
_TK = 32
_TM = 2048
_TB = 64
_NCLS = 18


def _knn_conv1_body(nseed, tq, wg_ref, rel_ref, sxyz_ref, feats_ref,
                    w1t_ref, b1_ref, h1_ref, stats_ref):
    b = pl.program_id(0)
    kt = pl.program_id(1)

    wg = wg_ref[0]
    rel = rel_ref[0]
    s = sxyz_ref[0]

    dx = wg[:, 0:1] - s[0:1, :]
    d2 = dx * dx
    dy = wg[:, 1:2] - s[1:2, :]
    d2 = d2 + dy * dy
    dz = wg[:, 2:3] - s[2:3, :]
    d2 = d2 + dz * dz

    acc = None
    wsum = None
    d2w = d2
    for _ in range(3):
        m = jnp.min(d2w, axis=1, keepdims=True)
        sel = d2w == m
        w = 1.0 / (jnp.sqrt(m) + 1e-8)
        acc = jnp.where(sel, w, 0.0 if acc is None else acc)
        wsum = w if wsum is None else wsum + w
        d2w = jnp.where(sel, jnp.float32(jnp.inf), d2w)

    interp = jnp.dot(acc, feats_ref[0], preferred_element_type=jnp.float32,
                     precision=jax.lax.Precision.HIGHEST)
    interp = interp * (1.0 / wsum)
    xcat = jnp.concatenate([rel, interp], axis=1)
    y = jnp.dot(xcat, w1t_ref[...], preferred_element_type=jnp.float32)
    y = y + b1_ref[:, :]
    h1_ref[0] = y

    @pl.when((b == 0) & (kt == 0))
    def _():
        stats_ref[:, :] = jnp.zeros_like(stats_ref)

    stats_ref[0:1, :] += jnp.sum(y, axis=0, keepdims=True)
    stats_ref[1:2, :] += jnp.sum(y * y, axis=0, keepdims=True)


def _bn_relu_mm_body(x_ref, sc_ref, sh_ref, wt_ref, b_ref, y_ref, stats_ref):
    i = pl.program_id(0)
    h = jnp.maximum(x_ref[...] * sc_ref[:, :] + sh_ref[:, :], 0.0)
    y = jnp.dot(h, wt_ref[...], preferred_element_type=jnp.float32)
    y = y + b_ref[:, :]
    y_ref[...] = y

    @pl.when(i == 0)
    def _():
        stats_ref[:, :] = jnp.zeros_like(stats_ref)

    stats_ref[0:1, :] += jnp.sum(y, axis=0, keepdims=True)
    stats_ref[1:2, :] += jnp.sum(y * y, axis=0, keepdims=True)


def _pool_conv_body(x_ref, sc_ref, sh_ref, wt_ref, b_ref, y_ref, stats_ref):
    i = pl.program_id(0)
    h = jnp.maximum(x_ref[...] * sc_ref[:, :] + sh_ref[:, :], 0.0)
    pooled = jnp.max(h, axis=1)
    y = jnp.dot(pooled, wt_ref[...], preferred_element_type=jnp.float32)
    y = y + b_ref[:, :]
    y_ref[...] = y

    @pl.when(i == 0)
    def _():
        stats_ref[:, :] = jnp.zeros_like(stats_ref)

    stats_ref[0:1, :] += jnp.sum(y, axis=0, keepdims=True)
    stats_ref[1:2, :] += jnp.sum(y * y, axis=0, keepdims=True)


def _head_body(x_ref, sc1_ref, sh1_ref, w2t_ref, b2_ref, g2_ref, be2_ref,
               w3t_ref, b3_ref, y_ref):
    h = jnp.maximum(x_ref[...] * sc1_ref[:, :] + sh1_ref[:, :], 0.0)
    c2 = jnp.dot(h, w2t_ref[...], preferred_element_type=jnp.float32)
    c2 = c2 + b2_ref[:, :]
    m = jnp.mean(c2, axis=0, keepdims=True)
    v = jnp.mean(c2 * c2, axis=0, keepdims=True) - m * m
    h2 = (c2 - m) / jnp.sqrt(v + 1e-5) * g2_ref[:, :] + be2_ref[:, :]
    h2 = jnp.maximum(h2, 0.0)
    y = jnp.dot(h2, w3t_ref[...], preferred_element_type=jnp.float32)
    y_ref[...] = y + b3_ref[:, :]


def _affine(stats, count, g, be, eps=1e-5):
    mean = stats[0] / count
    var = stats[1] / count - mean * mean
    sc = g / jnp.sqrt(var + eps)
    sh = be - mean * sc
    return sc.reshape(1, -1), sh.reshape(1, -1)


def kernel(center, size, heading, seed_xyz, seed_features, W1, b1, g1, be1,
           W2, b2, g2, be2, W3, b3, g3, be3, Wc1, bc1, gc1, bec1,
           Wc2, bc2, gc2, bec2, Wc3, bc3):
    B, K = size.shape[:2]
    N = seed_xyz.shape[1]
    C = seed_features.shape[1]
    Q = K * _G
    TQ = _TK * _G
    F = W1.shape[0]

    grid_step = jnp.linspace(-1.0, 1.0, _GS)
    gx = jnp.broadcast_to(grid_step.reshape(_GS, 1, 1), (_GS, _GS, _GS)).reshape(1, 1, _G)
    gy = jnp.broadcast_to(grid_step.reshape(1, _GS, 1), (_GS, _GS, _GS)).reshape(1, 1, _G)
    gz = jnp.broadcast_to(grid_step.reshape(1, 1, _GS), (_GS, _GS, _GS)).reshape(1, 1, _G)
    x_grid = gx * size[:, :, 0:1]
    y_grid = gy * size[:, :, 1:2]
    z_grid = gz * size[:, :, 2:3]
    whole_grid = jnp.stack([x_grid, y_grid, z_grid], axis=-1)
    c = jnp.cos(heading)
    s = jnp.sin(heading)
    z0 = jnp.zeros_like(c)
    o1 = jnp.ones_like(c)
    rot = jnp.stack([jnp.stack([c, -s, z0], axis=-1),
                     jnp.stack([s, c, z0], axis=-1),
                     jnp.stack([z0, z0, o1], axis=-1)], axis=-2)
    rel_grid = jnp.einsum('bkgi,bkji->bkgj', whole_grid, rot)
    whole_grid = rel_grid + center[:, :, None, :]
    wg = whole_grid.reshape(B, Q, 3)
    rel = rel_grid.reshape(B, Q, 3)

    sxyz_t = jnp.transpose(seed_xyz, (0, 2, 1))
    feats_t = jnp.transpose(seed_features, (0, 2, 1))
    w1t = jnp.transpose(W1, (1, 0))

    kt_steps = K // _TK
    h1, stats1 = pl.pallas_call(
        functools.partial(_knn_conv1_body, N, TQ),
        grid=(B, kt_steps),
        in_specs=[
            pl.BlockSpec((1, TQ, 3), lambda b, k: (b, k, 0)),
            pl.BlockSpec((1, TQ, 3), lambda b, k: (b, k, 0)),
            pl.BlockSpec((1, 3, N), lambda b, k: (b, 0, 0)),
            pl.BlockSpec((1, N, C), lambda b, k: (b, 0, 0)),
            pl.BlockSpec((3 + C, F), lambda b, k: (0, 0)),
            pl.BlockSpec((1, F), lambda b, k: (0, 0)),
        ],
        out_specs=[
            pl.BlockSpec((1, TQ, F), lambda b, k: (b, k, 0)),
            pl.BlockSpec((2, F), lambda b, k: (0, 0)),
        ],
        out_shape=[
            jax.ShapeDtypeStruct((B, Q, F), jnp.float32),
            jax.ShapeDtypeStruct((2, F), jnp.float32),
        ],
        compiler_params=pltpu.CompilerParams(
            dimension_semantics=("arbitrary", "arbitrary")),
    )(wg, rel, sxyz_t, feats_t, w1t, b1.reshape(1, F))

    BQ = B * Q
    h1 = h1.reshape(BQ, F)
    sc1, sh1 = _affine(stats1, BQ, g1, be1)

    def bn_relu_mm(x, sc, sh, wt, bias):
        rows = x.shape[0]
        return pl.pallas_call(
            _bn_relu_mm_body,
            grid=(rows // _TM,),
            in_specs=[
                pl.BlockSpec((_TM, F), lambda i: (i, 0)),
                pl.BlockSpec((1, F), lambda i: (0, 0)),
                pl.BlockSpec((1, F), lambda i: (0, 0)),
                pl.BlockSpec((F, F), lambda i: (0, 0)),
                pl.BlockSpec((1, F), lambda i: (0, 0)),
            ],
            out_specs=[
                pl.BlockSpec((_TM, F), lambda i: (i, 0)),
                pl.BlockSpec((2, F), lambda i: (0, 0)),
            ],
            out_shape=[
                jax.ShapeDtypeStruct((rows, F), jnp.float32),
                jax.ShapeDtypeStruct((2, F), jnp.float32),
            ],
            compiler_params=pltpu.CompilerParams(
                dimension_semantics=("arbitrary",)),
        )(x, sc, sh, wt, bias)

    h2, stats2 = bn_relu_mm(h1, sc1, sh1, jnp.transpose(W2, (1, 0)),
                            b2.reshape(1, F))
    sc2, sh2 = _affine(stats2, BQ, g2, be2)
    h3, stats3 = bn_relu_mm(h2, sc2, sh2, jnp.transpose(W3, (1, 0)),
                            b3.reshape(1, F))
    sc3, sh3 = _affine(stats3, BQ, g3, be3)

    BK = B * K
    h3g = h3.reshape(BK, _G, F)
    c1, statsc1 = pl.pallas_call(
        _pool_conv_body,
        grid=(BK // _TB,),
        in_specs=[
            pl.BlockSpec((_TB, _G, F), lambda i: (i, 0, 0)),
            pl.BlockSpec((1, F), lambda i: (0, 0)),
            pl.BlockSpec((1, F), lambda i: (0, 0)),
            pl.BlockSpec((F, F), lambda i: (0, 0)),
            pl.BlockSpec((1, F), lambda i: (0, 0)),
        ],
        out_specs=[
            pl.BlockSpec((_TB, F), lambda i: (i, 0)),
            pl.BlockSpec((2, F), lambda i: (0, 0)),
        ],
        out_shape=[
            jax.ShapeDtypeStruct((BK, F), jnp.float32),
            jax.ShapeDtypeStruct((2, F), jnp.float32),
        ],
        compiler_params=pltpu.CompilerParams(
            dimension_semantics=("arbitrary",)),
    )(h3g, sc3, sh3, jnp.transpose(Wc1, (1, 0)), bc1.reshape(1, F))

    scc1, shc1 = _affine(statsc1, BK, gc1, bec1)

    w3t = jnp.transpose(Wc3[-_NCLS:], (1, 0))
    out = pl.pallas_call(
        _head_body,
        grid=(1,),
        in_specs=[
            pl.BlockSpec((BK, F), lambda i: (0, 0)),
            pl.BlockSpec((1, F), lambda i: (0, 0)),
            pl.BlockSpec((1, F), lambda i: (0, 0)),
            pl.BlockSpec((F, F), lambda i: (0, 0)),
            pl.BlockSpec((1, F), lambda i: (0, 0)),
            pl.BlockSpec((1, F), lambda i: (0, 0)),
            pl.BlockSpec((1, F), lambda i: (0, 0)),
            pl.BlockSpec((F, _NCLS), lambda i: (0, 0)),
            pl.BlockSpec((1, _NCLS), lambda i: (0, 0)),
        ],
        out_specs=pl.BlockSpec((BK, _NCLS), lambda i: (0, 0)),
        out_shape=jax.ShapeDtypeStruct((BK, _NCLS), jnp.float32),
    )(c1, scc1, shc1, jnp.transpose(Wc2, (1, 0)), bc2.reshape(1, F),
      gc2.reshape(1, F), bec2.reshape(1, F), w3t, bc3[-_NCLS:].reshape(1, _NCLS))

    return out.reshape(B, K, _NCLS)


# speedup vs baseline: 1.1682x; 1.0584x over previous
import functools

import jax
import jax.numpy as jnp
from jax.experimental import pallas as pl
from jax.experimental.pallas import tpu as pltpu

_GS = 4
_G = _GS * _GS * _GS
_TK = 32
_TM = 8192
_TB = 128
_NCLS = 18


def _knn_conv1_body(nseed, tq, wg_ref, rel_ref, sxyz_ref, feats_ref,
                    w1t_ref, b1_ref, h1_ref, stats_ref):
    b = pl.program_id(0)
    kt = pl.program_id(1)

    wg = wg_ref[0]
    rel = rel_ref[0]
    s = sxyz_ref[0]

    dx = wg[:, 0:1] - s[0:1, :]
    d2 = dx * dx
    dy = wg[:, 1:2] - s[1:2, :]
    d2 = d2 + dy * dy
    dz = wg[:, 2:3] - s[2:3, :]
    d2 = d2 + dz * dz

    acc = None
    wsum = None
    d2w = d2
    for _ in range(3):
        m = jnp.min(d2w, axis=1, keepdims=True)
        sel = d2w == m
        w = 1.0 / (jnp.sqrt(m) + 1e-8)
        acc = jnp.where(sel, w, 0.0 if acc is None else acc)
        wsum = w if wsum is None else wsum + w
        d2w = jnp.where(sel, jnp.float32(jnp.inf), d2w)

    interp = jnp.dot(acc, feats_ref[0], preferred_element_type=jnp.float32,
                     precision=jax.lax.Precision.HIGHEST)
    interp = interp * (1.0 / wsum)
    xcat = jnp.concatenate([rel, interp], axis=1)
    y = jnp.dot(xcat, w1t_ref[...], preferred_element_type=jnp.float32)
    y = y + b1_ref[:, :]
    h1_ref[0] = y

    @pl.when((b == 0) & (kt == 0))
    def _():
        stats_ref[:, :] = jnp.zeros_like(stats_ref)

    stats_ref[0:1, :] += jnp.sum(y, axis=0, keepdims=True)
    stats_ref[1:2, :] += jnp.sum(y * y, axis=0, keepdims=True)


def _bn_relu_mm_body(x_ref, sc_ref, sh_ref, wt_ref, b_ref, y_ref, stats_ref):
    i = pl.program_id(0)
    h = jnp.maximum(x_ref[...] * sc_ref[:, :] + sh_ref[:, :], 0.0)
    y = jnp.dot(h, wt_ref[...], preferred_element_type=jnp.float32)
    y = y + b_ref[:, :]
    y_ref[...] = y

    @pl.when(i == 0)
    def _():
        stats_ref[:, :] = jnp.zeros_like(stats_ref)

    stats_ref[0:1, :] += jnp.sum(y, axis=0, keepdims=True)
    stats_ref[1:2, :] += jnp.sum(y * y, axis=0, keepdims=True)


def _pool_conv_body(x_ref, sc_ref, sh_ref, wt_ref, b_ref, y_ref, stats_ref):
    i = pl.program_id(0)
    h = jnp.maximum(x_ref[...] * sc_ref[:, :] + sh_ref[:, :], 0.0)
    pooled = jnp.max(h, axis=1)
    y = jnp.dot(pooled, wt_ref[...], preferred_element_type=jnp.float32)
    y = y + b_ref[:, :]
    y_ref[...] = y

    @pl.when(i == 0)
    def _():
        stats_ref[:, :] = jnp.zeros_like(stats_ref)

    stats_ref[0:1, :] += jnp.sum(y, axis=0, keepdims=True)
    stats_ref[1:2, :] += jnp.sum(y * y, axis=0, keepdims=True)


def _head_body(x_ref, sc1_ref, sh1_ref, w2t_ref, b2_ref, g2_ref, be2_ref,
               w3t_ref, b3_ref, y_ref):
    h = jnp.maximum(x_ref[...] * sc1_ref[:, :] + sh1_ref[:, :], 0.0)
    c2 = jnp.dot(h, w2t_ref[...], preferred_element_type=jnp.float32)
    c2 = c2 + b2_ref[:, :]
    m = jnp.mean(c2, axis=0, keepdims=True)
    v = jnp.mean(c2 * c2, axis=0, keepdims=True) - m * m
    h2 = (c2 - m) / jnp.sqrt(v + 1e-5) * g2_ref[:, :] + be2_ref[:, :]
    h2 = jnp.maximum(h2, 0.0)
    y = jnp.dot(h2, w3t_ref[...], preferred_element_type=jnp.float32)
    y_ref[...] = y + b3_ref[:, :]


def _affine(stats, count, g, be, eps=1e-5):
    mean = stats[0] / count
    var = stats[1] / count - mean * mean
    sc = g / jnp.sqrt(var + eps)
    sh = be - mean * sc
    return sc.reshape(1, -1), sh.reshape(1, -1)


def kernel(center, size, heading, seed_xyz, seed_features, W1, b1, g1, be1,
           W2, b2, g2, be2, W3, b3, g3, be3, Wc1, bc1, gc1, bec1,
           Wc2, bc2, gc2, bec2, Wc3, bc3):
    B, K = size.shape[:2]
    N = seed_xyz.shape[1]
    C = seed_features.shape[1]
    Q = K * _G
    TQ = _TK * _G
    F = W1.shape[0]

    grid_step = jnp.linspace(-1.0, 1.0, _GS)
    gx = jnp.broadcast_to(grid_step.reshape(_GS, 1, 1), (_GS, _GS, _GS)).reshape(1, 1, _G)
    gy = jnp.broadcast_to(grid_step.reshape(1, _GS, 1), (_GS, _GS, _GS)).reshape(1, 1, _G)
    gz = jnp.broadcast_to(grid_step.reshape(1, 1, _GS), (_GS, _GS, _GS)).reshape(1, 1, _G)
    x_grid = gx * size[:, :, 0:1]
    y_grid = gy * size[:, :, 1:2]
    z_grid = gz * size[:, :, 2:3]
    whole_grid = jnp.stack([x_grid, y_grid, z_grid], axis=-1)
    c = jnp.cos(heading)
    s = jnp.sin(heading)
    z0 = jnp.zeros_like(c)
    o1 = jnp.ones_like(c)
    rot = jnp.stack([jnp.stack([c, -s, z0], axis=-1),
                     jnp.stack([s, c, z0], axis=-1),
                     jnp.stack([z0, z0, o1], axis=-1)], axis=-2)
    rel_grid = jnp.einsum('bkgi,bkji->bkgj', whole_grid, rot)
    whole_grid = rel_grid + center[:, :, None, :]
    wg = whole_grid.reshape(B, Q, 3)
    rel = rel_grid.reshape(B, Q, 3)

    sxyz_t = jnp.transpose(seed_xyz, (0, 2, 1))
    feats_t = jnp.transpose(seed_features, (0, 2, 1))
    w1t = jnp.transpose(W1, (1, 0))

    kt_steps = K // _TK
    h1, stats1 = pl.pallas_call(
        functools.partial(_knn_conv1_body, N, TQ),
        grid=(B, kt_steps),
        in_specs=[
            pl.BlockSpec((1, TQ, 3), lambda b, k: (b, k, 0)),
            pl.BlockSpec((1, TQ, 3), lambda b, k: (b, k, 0)),
            pl.BlockSpec((1, 3, N), lambda b, k: (b, 0, 0)),
            pl.BlockSpec((1, N, C), lambda b, k: (b, 0, 0)),
            pl.BlockSpec((3 + C, F), lambda b, k: (0, 0)),
            pl.BlockSpec((1, F), lambda b, k: (0, 0)),
        ],
        out_specs=[
            pl.BlockSpec((1, TQ, F), lambda b, k: (b, k, 0)),
            pl.BlockSpec((2, F), lambda b, k: (0, 0)),
        ],
        out_shape=[
            jax.ShapeDtypeStruct((B, Q, F), jnp.float32),
            jax.ShapeDtypeStruct((2, F), jnp.float32),
        ],
        compiler_params=pltpu.CompilerParams(
            dimension_semantics=("arbitrary", "arbitrary")),
    )(wg, rel, sxyz_t, feats_t, w1t, b1.reshape(1, F))

    BQ = B * Q
    h1 = h1.reshape(BQ, F)
    sc1, sh1 = _affine(stats1, BQ, g1, be1)

    def bn_relu_mm(x, sc, sh, wt, bias):
        rows = x.shape[0]
        return pl.pallas_call(
            _bn_relu_mm_body,
            grid=(rows // _TM,),
            in_specs=[
                pl.BlockSpec((_TM, F), lambda i: (i, 0)),
                pl.BlockSpec((1, F), lambda i: (0, 0)),
                pl.BlockSpec((1, F), lambda i: (0, 0)),
                pl.BlockSpec((F, F), lambda i: (0, 0)),
                pl.BlockSpec((1, F), lambda i: (0, 0)),
            ],
            out_specs=[
                pl.BlockSpec((_TM, F), lambda i: (i, 0)),
                pl.BlockSpec((2, F), lambda i: (0, 0)),
            ],
            out_shape=[
                jax.ShapeDtypeStruct((rows, F), jnp.float32),
                jax.ShapeDtypeStruct((2, F), jnp.float32),
            ],
            compiler_params=pltpu.CompilerParams(
                dimension_semantics=("arbitrary",)),
        )(x, sc, sh, wt, bias)

    h2, stats2 = bn_relu_mm(h1, sc1, sh1, jnp.transpose(W2, (1, 0)),
                            b2.reshape(1, F))
    sc2, sh2 = _affine(stats2, BQ, g2, be2)
    h3, stats3 = bn_relu_mm(h2, sc2, sh2, jnp.transpose(W3, (1, 0)),
                            b3.reshape(1, F))
    sc3, sh3 = _affine(stats3, BQ, g3, be3)

    BK = B * K
    h3g = h3.reshape(BK, _G, F)
    c1, statsc1 = pl.pallas_call(
        _pool_conv_body,
        grid=(BK // _TB,),
        in_specs=[
            pl.BlockSpec((_TB, _G, F), lambda i: (i, 0, 0)),
            pl.BlockSpec((1, F), lambda i: (0, 0)),
            pl.BlockSpec((1, F), lambda i: (0, 0)),
            pl.BlockSpec((F, F), lambda i: (0, 0)),
            pl.BlockSpec((1, F), lambda i: (0, 0)),
        ],
        out_specs=[
            pl.BlockSpec((_TB, F), lambda i: (i, 0)),
            pl.BlockSpec((2, F), lambda i: (0, 0)),
        ],
        out_shape=[
            jax.ShapeDtypeStruct((BK, F), jnp.float32),
            jax.ShapeDtypeStruct((2, F), jnp.float32),
        ],
        compiler_params=pltpu.CompilerParams(
            dimension_semantics=("arbitrary",)),
    )(h3g, sc3, sh3, jnp.transpose(Wc1, (1, 0)), bc1.reshape(1, F))

    scc1, shc1 = _affine(statsc1, BK, gc1, bec1)

    w3t = jnp.transpose(Wc3[-_NCLS:], (1, 0))
    out = pl.pallas_call(
        _head_body,
        grid=(1,),
        in_specs=[
            pl.BlockSpec((BK, F), lambda i: (0, 0)),
            pl.BlockSpec((1, F), lambda i: (0, 0)),
            pl.BlockSpec((1, F), lambda i: (0, 0)),
            pl.BlockSpec((F, F), lambda i: (0, 0)),
            pl.BlockSpec((1, F), lambda i: (0, 0)),
            pl.BlockSpec((1, F), lambda i: (0, 0)),
            pl.BlockSpec((1, F), lambda i: (0, 0)),
            pl.BlockSpec((F, _NCLS), lambda i: (0, 0)),
            pl.BlockSpec((1, _NCLS), lambda i: (0, 0)),
        ],
        out_specs=pl.BlockSpec((BK, _NCLS), lambda i: (0, 0)),
        out_shape=jax.ShapeDtypeStruct((BK, _NCLS), jnp.float32),
    )(c1, scc1, shc1, jnp.transpose(Wc2, (1, 0)), bc2.reshape(1, F),
      gc2.reshape(1, F), bec2.reshape(1, F), w3t, bc3[-_NCLS:].reshape(1, _NCLS))

    return out.reshape(B, K, _NCLS)
